# baseline (speedup 1.0000x reference)
import functools

import jax
import jax.numpy as jnp
from jax import lax
from jax.experimental import pallas as pl
from jax.experimental.pallas import tpu as pltpu

N_ENT = 100000
N_REL = 100000
D = 32
B = 128
NSTREAM = 8
TILE = 3072
GRID = 4
REM = N_ENT - NSTREAM * TILE * GRID


def _score_body(hrt_ref, entT_hbm, relT_hbm, hrtv_ref, *refs):
    ent_refs = refs[:NSTREAM]
    out_ref = refs[NSTREAM]
    gtile_ref, rem_ref, q_ref, thr_ref, acc_ref, sem = refs[NSTREAM + 1:]
    i = pl.program_id(0)

    @pl.when(i == 0)
    def _():
        pltpu.make_async_copy(
            entT_hbm.at[:, pl.ds(N_ENT - REM, REM)], rem_ref, sem).start()

        def issue(j, _, tab_hbm, base):
            cb = pl.multiple_of((hrt_ref[base + j] // 128) * 128, 128)
            pltpu.make_async_copy(
                tab_hbm.at[:, pl.ds(cb, 128)],
                gtile_ref.at[base + j], sem).start()
            return 0

        lax.fori_loop(0, B, functools.partial(issue, tab_hbm=entT_hbm,
                                              base=0), 0)
        lax.fori_loop(0, B, functools.partial(issue, tab_hbm=relT_hbm,
                                              base=B), 0)
        lax.fori_loop(0, B, functools.partial(issue, tab_hbm=entT_hbm,
                                              base=2 * B), 0)

        def drain(j, _):
            pltpu.make_async_copy(
                entT_hbm.at[:, pl.ds(0, 128)],
                gtile_ref.at[j], sem).wait()
            return 0

        lax.fori_loop(0, 3 * B, drain, 0)
        pltpu.make_async_copy(
            entT_hbm.at[:, pl.ds(N_ENT - REM, REM)], rem_ref, sem).wait()

        lane = jax.lax.broadcasted_iota(jnp.int32, (3 * B, 1, 128), 2)
        want = (hrtv_ref[...] % 128).reshape(3 * B, 1, 1)
        maskf = jnp.where(lane == want, 1.0, 0.0)
        gath = jnp.sum(gtile_ref[...] * maskf, axis=2)
        eh = gath[0:B]
        er = gath[B:2 * B]
        et = gath[2 * B:3 * B]
        q_ref[...] = jnp.concatenate([er * et, eh * er],
                                     axis=0).T.astype(jnp.bfloat16)
        p2 = jnp.concatenate([eh * er * et, eh * er * et], axis=0)
        thr2 = jnp.sum(p2.T, axis=0, keepdims=True)
        thr_ref[...] = thr2
        s_rem = lax.dot_general(rem_ref[...].astype(jnp.bfloat16), q_ref[...],
                                (((0,), (0,)), ((), ())),
                                preferred_element_type=jnp.float32)
        acc_ref[...] = jnp.sum(jnp.where(s_rem > thr2, 1.0, 0.0),
                               axis=0, keepdims=True)

    @pl.when(i > 0)
    def _():
        q = q_ref[...]
        thr = thr_ref[...]
        a = None
        for ent_ref in ent_refs:
            s = lax.dot_general(ent_ref[...].astype(jnp.bfloat16), q,
                                (((0,), (0,)), ((), ())),
                                preferred_element_type=jnp.float32)
            c = jnp.sum(jnp.where(s > thr, 1.0, 0.0), axis=0, keepdims=True)
            a = c if a is None else a + c
        acc_ref[...] += a

    @pl.when(i == GRID)
    def _():
        out_ref[...] = acc_ref[...].astype(jnp.int32) + 1


def _score(hrt, entT, relT, interpret=False):
    grid_spec = pltpu.PrefetchScalarGridSpec(
        num_scalar_prefetch=1,
        grid=(GRID + 1,),
        in_specs=[
            pl.BlockSpec(memory_space=pl.ANY),
            pl.BlockSpec(memory_space=pl.ANY),
            pl.BlockSpec((3 * B, 1), lambda i, hrt: (0, 0)),
        ] + [
            pl.BlockSpec((D, TILE), functools.partial(
                lambda i, hrt, k: (0, k * GRID + jnp.maximum(i - 1, 0)), k=k))
            for k in range(NSTREAM)
        ],
        out_specs=pl.BlockSpec((1, 2 * B), lambda i, hrt: (0, 0)),
        scratch_shapes=[
            pltpu.VMEM((3 * B, D, 128), jnp.float32),
            pltpu.VMEM((D, REM), jnp.float32),
            pltpu.VMEM((D, 2 * B), jnp.bfloat16),
            pltpu.VMEM((1, 2 * B), jnp.float32),
            pltpu.VMEM((1, 2 * B), jnp.float32),
            pltpu.SemaphoreType.DMA,
        ],
    )
    return pl.pallas_call(
        _score_body,
        grid_spec=grid_spec,
        out_shape=jax.ShapeDtypeStruct((1, 2 * B), jnp.int32),
        compiler_params=pltpu.CompilerParams(
            dimension_semantics=("arbitrary",)),
        interpret=interpret,
    )(hrt, entT, relT, hrt.reshape(3 * B, 1), *([entT] * NSTREAM))


def kernel(raw_triples, entity_emb, rel_emb):
    trip = raw_triples.astype(jnp.int32)
    hrt = jnp.concatenate([trip[:, 0] % N_ENT,
                           trip[:, 1] % N_REL,
                           trip[:, 2] % N_ENT])
    cnt = _score(hrt, entity_emb.T, rel_emb.T)
    return (cnt[0, :B], cnt[0, B:])


# speedup vs baseline: 1.0004x; 1.0004x over previous
import functools

import jax
import jax.numpy as jnp
from jax import lax
from jax.experimental import pallas as pl
from jax.experimental.pallas import tpu as pltpu

N_ENT = 100000
N_REL = 100000
D = 32
B = 128
NSTREAM = 8
TILE = 3072
GRID = 4
REM = N_ENT - NSTREAM * TILE * GRID


def _score_body(hrt_ref, entT_hbm, relT_hbm, hrtv_ref, *refs):
    ent_refs = refs[:NSTREAM]
    out_ref = refs[NSTREAM]
    gtile_ref, rem_ref, q_ref, thr_ref, acc_ref, sem = refs[NSTREAM + 1:]
    i = pl.program_id(0)

    @pl.when(i == 0)
    def _():
        pltpu.make_async_copy(
            entT_hbm.at[:, pl.ds(N_ENT - REM, REM)], rem_ref, sem).start()

        def issue(j, _, tab_hbm, base):
            cb = pl.multiple_of((hrt_ref[base + j] // 128) * 128, 128)
            pltpu.make_async_copy(
                tab_hbm.at[:, pl.ds(cb, 128)],
                gtile_ref.at[base + j], sem).start()
            return 0

        lax.fori_loop(0, B, functools.partial(issue, tab_hbm=entT_hbm,
                                              base=0), 0)
        lax.fori_loop(0, B, functools.partial(issue, tab_hbm=relT_hbm,
                                              base=B), 0)
        lax.fori_loop(0, B, functools.partial(issue, tab_hbm=entT_hbm,
                                              base=2 * B), 0)

        def drain(j, _):
            pltpu.make_async_copy(
                entT_hbm.at[:, pl.ds(0, 128)],
                gtile_ref.at[j], sem).wait()
            return 0

        lax.fori_loop(0, 3 * B, drain, 0)
        pltpu.make_async_copy(
            entT_hbm.at[:, pl.ds(N_ENT - REM, REM)], rem_ref, sem).wait()

        lane = jax.lax.broadcasted_iota(jnp.int32, (3 * B, 1, 128), 2)
        want = (hrtv_ref[...] % 128).reshape(3 * B, 1, 1)
        maskf = jnp.where(lane == want, 1.0, 0.0)
        gath = jnp.sum(gtile_ref[...] * maskf, axis=2)
        eh = gath[0:B]
        er = gath[B:2 * B]
        et = gath[2 * B:3 * B]
        q_ref[...] = jnp.concatenate([er * et, eh * er],
                                     axis=0).astype(jnp.bfloat16)
        t1 = jnp.sum(eh * er * et, axis=1, keepdims=True)
        thr2 = jnp.concatenate([t1, t1], axis=0)
        thr_ref[...] = thr2
        s_rem = lax.dot_general(q_ref[...], rem_ref[...].astype(jnp.bfloat16),
                                (((1,), (0,)), ((), ())),
                                preferred_element_type=jnp.float32)
        acc_ref[...] = jnp.sum(jnp.where(s_rem > thr2, 1.0, 0.0),
                               axis=1, keepdims=True)

    q = q_ref[...]
    thr = thr_ref[...]
    a = None
    for ent_ref in ent_refs:
        s = lax.dot_general(q, ent_ref[...].astype(jnp.bfloat16),
                            (((1,), (0,)), ((), ())),
                            preferred_element_type=jnp.float32)
        c = jnp.sum(jnp.where(s > thr, 1.0, 0.0), axis=1, keepdims=True)
        a = c if a is None else a + c
    acc_ref[...] += a

    @pl.when(i == GRID - 1)
    def _():
        out_ref[...] = acc_ref[...].astype(jnp.int32) + 1


def _score(hrt, entT, relT, interpret=False):
    grid_spec = pltpu.PrefetchScalarGridSpec(
        num_scalar_prefetch=1,
        grid=(GRID,),
        in_specs=[
            pl.BlockSpec(memory_space=pl.ANY),
            pl.BlockSpec(memory_space=pl.ANY),
            pl.BlockSpec((3 * B, 1), lambda i, hrt: (0, 0)),
        ] + [
            pl.BlockSpec((D, TILE), functools.partial(
                lambda i, hrt, k: (0, k * GRID + i), k=k))
            for k in range(NSTREAM)
        ],
        out_specs=pl.BlockSpec((2 * B, 1), lambda i, hrt: (0, 0)),
        scratch_shapes=[
            pltpu.VMEM((3 * B, D, 128), jnp.float32),
            pltpu.VMEM((D, REM), jnp.float32),
            pltpu.VMEM((2 * B, D), jnp.bfloat16),
            pltpu.VMEM((2 * B, 1), jnp.float32),
            pltpu.VMEM((2 * B, 1), jnp.float32),
            pltpu.SemaphoreType.DMA,
        ],
    )
    return pl.pallas_call(
        _score_body,
        grid_spec=grid_spec,
        out_shape=jax.ShapeDtypeStruct((2 * B, 1), jnp.int32),
        compiler_params=pltpu.CompilerParams(
            dimension_semantics=("arbitrary",)),
        interpret=interpret,
    )(hrt, entT, relT, hrt.reshape(3 * B, 1), *([entT] * NSTREAM))


def kernel(raw_triples, entity_emb, rel_emb):
    trip = raw_triples.astype(jnp.int32)
    hrt = jnp.concatenate([trip[:, 0] % N_ENT,
                           trip[:, 1] % N_REL,
                           trip[:, 2] % N_ENT])
    cnt = _score(hrt, entity_emb.T, rel_emb.T)
    return (cnt[:B, 0], cnt[B:, 0])


# speedup vs baseline: 1.0103x; 1.0099x over previous
import functools

import jax
import jax.numpy as jnp
from jax import lax
from jax.experimental import pallas as pl
from jax.experimental.pallas import tpu as pltpu

N_ENT = 100000
N_REL = 100000
D = 32
B = 128
NSTREAM = 8
TILE = 3072
GRID = 4
REM = N_ENT - NSTREAM * TILE * GRID


def _score_body(hrt_ref, entT_hbm, relT_hbm, hrtv_ref, *refs):
    ent_refs = refs[:NSTREAM]
    out_ref = refs[NSTREAM]
    gtile_ref, rem_ref, q_ref, thr_ref, acc_ref, sem = refs[NSTREAM + 1:]
    i = pl.program_id(0)

    @pl.when(i == 0)
    def _():
        pltpu.make_async_copy(
            entT_hbm.at[:, pl.ds(N_ENT - REM, REM)], rem_ref, sem).start()

        def issue(j, _, tab_hbm, base):
            cb = pl.multiple_of((hrt_ref[base + j] // 128) * 128, 128)
            pltpu.make_async_copy(
                tab_hbm.at[:, pl.ds(cb, 128)],
                gtile_ref.at[base + j], sem).start()
            return 0

        lax.fori_loop(0, B, functools.partial(issue, tab_hbm=entT_hbm,
                                              base=0), 0)
        lax.fori_loop(0, B, functools.partial(issue, tab_hbm=relT_hbm,
                                              base=B), 0)
        lax.fori_loop(0, B, functools.partial(issue, tab_hbm=entT_hbm,
                                              base=2 * B), 0)

        def drain(j, _):
            pltpu.make_async_copy(
                entT_hbm.at[:, pl.ds(0, 128)],
                gtile_ref.at[j], sem).wait()
            return 0

        lax.fori_loop(0, 3 * B, drain, 0)
        pltpu.make_async_copy(
            entT_hbm.at[:, pl.ds(N_ENT - REM, REM)], rem_ref, sem).wait()

        lane = jax.lax.broadcasted_iota(jnp.int32, (3 * B, 1, 128), 2)
        want = (hrtv_ref[...] % 128).reshape(3 * B, 1, 1)
        maskf = jnp.where(lane == want, 1.0, 0.0)
        gath = jnp.sum(gtile_ref[...] * maskf, axis=2)
        eh = gath[0:B]
        er = gath[B:2 * B]
        et = gath[2 * B:3 * B]
        q_ref[...] = jnp.concatenate([er * et, eh * er],
                                     axis=0).T.astype(jnp.bfloat16)
        p2 = jnp.concatenate([eh * er * et, eh * er * et], axis=0)
        thr2 = jnp.sum(p2.T, axis=0, keepdims=True)
        thr_ref[...] = thr2
        s_rem = lax.dot_general(rem_ref[...].astype(jnp.bfloat16), q_ref[...],
                                (((0,), (0,)), ((), ())),
                                preferred_element_type=jnp.float32)
        acc_ref[...] = jnp.sum(jnp.where(s_rem > thr2, 1.0, 0.0),
                               axis=0, keepdims=True)

    q = q_ref[...]
    thr = thr_ref[...]
    a = None
    for ent_ref in ent_refs:
        s = lax.dot_general(ent_ref[...].astype(jnp.bfloat16), q,
                            (((0,), (0,)), ((), ())),
                            preferred_element_type=jnp.float32)
        c = jnp.sum(jnp.where(s > thr, 1.0, 0.0), axis=0, keepdims=True)
        a = c if a is None else a + c
    acc_ref[...] += a

    @pl.when(i == GRID - 1)
    def _():
        out_ref[...] = acc_ref[...].astype(jnp.int32) + 1


def _score(hrt, entT, relT, interpret=False):
    grid_spec = pltpu.PrefetchScalarGridSpec(
        num_scalar_prefetch=1,
        grid=(GRID,),
        in_specs=[
            pl.BlockSpec(memory_space=pl.ANY),
            pl.BlockSpec(memory_space=pl.ANY),
            pl.BlockSpec((3 * B, 1), lambda i, hrt: (0, 0)),
        ] + [
            pl.BlockSpec((D, TILE), functools.partial(
                lambda i, hrt, k: (0, k * GRID + i), k=k))
            for k in range(NSTREAM)
        ],
        out_specs=pl.BlockSpec((1, 2 * B), lambda i, hrt: (0, 0)),
        scratch_shapes=[
            pltpu.VMEM((3 * B, D, 128), jnp.float32),
            pltpu.VMEM((D, REM), jnp.float32),
            pltpu.VMEM((D, 2 * B), jnp.bfloat16),
            pltpu.VMEM((1, 2 * B), jnp.float32),
            pltpu.VMEM((1, 2 * B), jnp.float32),
            pltpu.SemaphoreType.DMA,
        ],
    )
    return pl.pallas_call(
        _score_body,
        grid_spec=grid_spec,
        out_shape=jax.ShapeDtypeStruct((1, 2 * B), jnp.int32),
        compiler_params=pltpu.CompilerParams(
            dimension_semantics=("arbitrary",)),
        interpret=interpret,
    )(hrt, entT, relT, hrt.reshape(3 * B, 1), *([entT] * NSTREAM))


def kernel(raw_triples, entity_emb, rel_emb):
    trip = raw_triples.astype(jnp.int32)
    hrt = jnp.concatenate([trip[:, 0] % N_ENT,
                           trip[:, 1] % N_REL,
                           trip[:, 2] % N_ENT])
    cnt = _score(hrt, entity_emb.T, rel_emb.T)
    return (cnt[0, :B], cnt[0, B:])
